# baseline (speedup 1.0000x reference)
import functools

import jax
import jax.numpy as jnp
from jax import lax
from jax.experimental import pallas as pl
from jax.experimental.pallas import tpu as pltpu
from jax.experimental.pallas import tpu_sc as plsc

_NUM_CORES = 2
_NUM_SUBCORES = 16
_NW = _NUM_CORES * _NUM_SUBCORES


def _ring_gather(table_hbm, out_hbm, idx_v, rows_v, sg, sw, base, b_per_w):
  nbuf, chunk, _ = rows_v.shape
  assert b_per_w % chunk == 0 and chunk % 8 == 0
  num_chunks = b_per_w // chunk
  assert num_chunks >= nbuf
  main_iters = (num_chunks - nbuf) // nbuf
  rem_lo = main_iters * nbuf

  def g_issue(c, b):
    pltpu.make_async_copy(
        table_hbm.at[idx_v.at[pl.ds(c * chunk, chunk)]], rows_v.at[b], sg[b]
    ).start()

  def g_wait(b):
    pltpu.make_async_copy(
        table_hbm.at[idx_v.at[pl.ds(0, chunk)]], rows_v.at[b], sg[b]
    ).wait()

  def w_issue(c, b):
    pltpu.make_async_copy(
        rows_v.at[b], out_hbm.at[pl.ds(base + c * chunk, chunk)], sw[b]
    ).start()

  def w_wait(b):
    pltpu.make_async_copy(
        rows_v.at[b], out_hbm.at[pl.ds(base, chunk)], sw[b]
    ).wait()

  for b in range(nbuf):
    g_issue(b, b)

  def body(i, carry):
    c0 = i * nbuf
    for b in range(nbuf):
      g_wait(b)
      w_issue(c0 + b, b)
    for b in range(nbuf):
      w_wait(b)
      g_issue(c0 + nbuf + b, b)
    return carry

  lax.fori_loop(0, main_iters, body, 0)

  for c in range(rem_lo, num_chunks):
    b = c % nbuf
    g_wait(b)
    w_issue(c, b)
    if c + nbuf < num_chunks:
      w_wait(b)
      g_issue(c + nbuf, b)
  for b in range(nbuf):
    w_wait(b)


def _make_dual_gather(dim, n0, n1, chunk, nbuf):
  assert n0 % _NW == 0 and n1 % _NW == 0
  bw0 = n0 // _NW
  bw1 = n1 // _NW
  mesh = plsc.VectorSubcoreMesh(core_axis_name="c", subcore_axis_name="s")

  @functools.partial(
      pl.kernel,
      mesh=mesh,
      compiler_params=pltpu.CompilerParams(needs_layout_passes=False),
      out_type=(
          jax.ShapeDtypeStruct((n0, dim), jnp.float32),
          jax.ShapeDtypeStruct((2 * n1, dim), jnp.float32),
      ),
      scratch_types=[
          pltpu.VMEM((bw0,), jnp.int32),
          pltpu.VMEM((bw1,), jnp.int32),
          pltpu.VMEM((2 * bw1,), jnp.int32),
          pltpu.VMEM((nbuf, chunk, dim), jnp.float32),
          [pltpu.SemaphoreType.DMA] * nbuf,
          [pltpu.SemaphoreType.DMA] * nbuf,
          [pltpu.SemaphoreType.DMA] * nbuf,
          [pltpu.SemaphoreType.DMA] * nbuf,
      ],
  )
  def dual(t0_hbm, i0_hbm, t1v_hbm, i1_hbm, o0_hbm, o1v_hbm,
           idx0_v, idx1_v, virt_v, rows_v, sg0, sw0, sg1, sw1):
    wid = lax.axis_index("s") * _NUM_CORES + lax.axis_index("c")
    base0 = wid * bw0
    base1 = wid * bw1
    pltpu.sync_copy(i0_hbm.at[pl.ds(base0, bw0)], idx0_v)
    pltpu.sync_copy(i1_hbm.at[pl.ds(base1, bw1)], idx1_v)

    lanes = lax.iota(jnp.int32, 16)

    def build(k, carry):
      src = k * 16 + lanes
      x = plsc.load_gather(idx1_v, [src])
      plsc.store_scatter(virt_v, [src * 2], x * 2)
      plsc.store_scatter(virt_v, [src * 2 + 1], x * 2 + 1)
      return carry

    lax.fori_loop(0, bw1 // 16, build, 0)

    _ring_gather(t0_hbm, o0_hbm, idx0_v, rows_v, sg0, sw0, base0, bw0)
    _ring_gather(t1v_hbm, o1v_hbm, virt_v, rows_v, sg1, sw1, 2 * base1,
                 2 * bw1)

  return dual


def kernel(captions_0, captions_1, from_table, to_table):
  b0, t0 = captions_0.shape
  b1, t1 = captions_1.shape
  n0 = b0 * t0
  n1 = b1 * t1
  fd = from_table.shape[1]
  tv, td = to_table.shape

  dual = _make_dual_gather(fd, n0, n1, chunk=16, nbuf=3)
  from_rows, to_rows_v = dual(
      from_table,
      captions_0.reshape(n0),
      to_table.reshape(2 * tv, td // 2),
      captions_1.reshape(n1),
  )
  return (from_rows.reshape(b0, t0, fd), to_rows_v.reshape(b1, t1, td))


# speedup vs baseline: 2.2365x; 2.2365x over previous
import functools

import jax
import jax.numpy as jnp
from jax import lax
from jax.experimental import pallas as pl
from jax.experimental.pallas import tpu as pltpu
from jax.experimental.pallas import tpu_sc as plsc

_NUM_CORES = 2
_NUM_SUBCORES = 16
_NW = _NUM_CORES * _NUM_SUBCORES


def _make_gather(dim, batch, chunk, nbuf):
  assert batch % _NW == 0
  b_per_w = batch // _NW
  assert b_per_w % chunk == 0 and chunk % 8 == 0
  num_chunks = b_per_w // chunk
  assert num_chunks >= nbuf
  main_iters = (num_chunks - nbuf) // nbuf
  rem_lo = main_iters * nbuf
  mesh = plsc.VectorSubcoreMesh(core_axis_name="c", subcore_axis_name="s")

  @functools.partial(
      pl.kernel,
      mesh=mesh,
      out_type=jax.ShapeDtypeStruct((batch, dim), jnp.float32),
      scratch_types=[
          pltpu.VMEM((b_per_w,), jnp.int32),
          pltpu.VMEM((nbuf, chunk, dim), jnp.float32),
          [pltpu.SemaphoreType.DMA] * nbuf,
          [pltpu.SemaphoreType.DMA] * nbuf,
      ],
  )
  def gather(table_hbm, idx_hbm, out_hbm, idx_v, rows_v, sg, sw):
    wid = lax.axis_index("s") * _NUM_CORES + lax.axis_index("c")
    base = wid * b_per_w
    pltpu.sync_copy(idx_hbm.at[pl.ds(base, b_per_w)], idx_v)

    def g_issue(c, b):
      pltpu.make_async_copy(
          table_hbm.at[idx_v.at[pl.ds(c * chunk, chunk)]], rows_v.at[b], sg[b]
      ).start()

    def g_wait(b):
      pltpu.make_async_copy(
          table_hbm.at[idx_v.at[pl.ds(0, chunk)]], rows_v.at[b], sg[b]
      ).wait()

    def w_issue(c, b):
      pltpu.make_async_copy(
          rows_v.at[b], out_hbm.at[pl.ds(base + c * chunk, chunk)], sw[b]
      ).start()

    def w_wait(b):
      pltpu.make_async_copy(
          rows_v.at[b], out_hbm.at[pl.ds(base, chunk)], sw[b]
      ).wait()

    for b in range(nbuf):
      g_issue(b, b)

    def body(i, carry):
      c0 = i * nbuf
      for b in range(nbuf):
        g_wait(b)
        w_issue(c0 + b, b)
      for b in range(nbuf):
        w_wait(b)
        g_issue(c0 + nbuf + b, b)
      return carry

    lax.fori_loop(0, main_iters, body, 0)

    for c in range(rem_lo, num_chunks):
      b = c % nbuf
      g_wait(b)
      w_issue(c, b)
      if c + nbuf < num_chunks:
        w_wait(b)
        g_issue(c + nbuf, b)
    for b in range(nbuf):
      w_wait(b)

  return gather


def kernel(captions_0, captions_1, from_table, to_table):
  b0, t0 = captions_0.shape
  b1, t1 = captions_1.shape
  n0 = b0 * t0
  n1 = b1 * t1
  fd = from_table.shape[1]
  td = to_table.shape[1]

  g0 = _make_gather(fd, n0, chunk=16, nbuf=3)
  g1 = _make_gather(td, n1, chunk=8, nbuf=3)

  from_rows = g0(from_table, captions_0.reshape(n0))
  to_rows = g1(to_table, captions_1.reshape(n1))
  return (from_rows.reshape(b0, t0, fd), to_rows.reshape(b1, t1, td))
